# Initial kernel scaffold; baseline (speedup 1.0000x reference)
import jax
import jax.numpy as jnp
from jax.experimental import pallas as pl


def kernel(x, edge_index, y, batch, W1, b1, W2, b2, W3, b3, fc1_W, fc1_b, fc2_W, fc2_b, fc3_W, fc3_b):
    raise NotImplementedError("write your pallas kernel here")


# speedup vs baseline: 28.3128x; 28.3128x over previous
import jax
import jax.numpy as jnp
from jax import lax
from jax.experimental import pallas as pl
from jax.experimental.pallas import tpu as pltpu
from jax.experimental.pallas import tpu_sc as plsc

N_NODES = 100000
N_EDGES = 3200000
F = 4
H = 16
G = 128
NC, NS = 2, 16
NW = NC * NS
CH = 128
IB = 16
OUTER = 49
PER_W = CH * IB * OUTER
EPAD = PER_W * NW
NP = 102400
RPT = NP // NS
BM = 4096


def _mesh():
    return plsc.VectorSubcoreMesh(core_axis_name="c", subcore_axis_name="s")


def _sc_deg_body(dst_hbm, out_hbm, didx, ones, acc, sem):
    del sem
    c = lax.axis_index("c")
    s = lax.axis_index("s")
    w = s * NC + c
    @pl.loop(0, 8)
    def _fill0(i):
        ones[pl.ds(i * 16, 16)] = jnp.zeros((16,), jnp.float32)

    @pl.loop(0, RPT // CH)
    def _zero(k):
        pltpu.sync_copy(ones, acc.at[pl.ds(s * RPT + k * CH, CH)])

    @pl.loop(0, 8)
    def _fill1(i):
        ones[pl.ds(i * 16, 16)] = jnp.ones((16,), jnp.float32)

    plsc.subcore_barrier()
    base = w * (PER_W // CH)
    @pl.loop(0, OUTER)
    def _outer(o):
        pltpu.sync_copy(dst_hbm.at[pl.ds(base + o * IB, IB)], didx)
        @pl.loop(0, IB)
        def _inner(j):
            pltpu.sync_copy(ones, acc.at[didx.at[j]], add=True)

    plsc.subcore_barrier()
    pltpu.sync_copy(acc.at[pl.ds(s * RPT, RPT)], out_hbm.at[c, pl.ds(s * RPT, RPT)])


def _sc_deg():
    return pl.kernel(
        _sc_deg_body,
        out_type=jax.ShapeDtypeStruct((NC, NP), jnp.float32),
        mesh=_mesh(),
        compiler_params=pltpu.CompilerParams(use_tc_tiling_on_sc=False),
        scratch_types=[
            pltpu.VMEM((IB, CH), jnp.int32),
            pltpu.VMEM((CH,), jnp.float32),
            pltpu.VMEM_SHARED((NP,), jnp.float32),
            pltpu.SemaphoreType.DMA,
        ],
    )


def _sc_prop_body(u_hbm, src_hbm, dst_hbm, out_hbm, sidx, didx, rows, acc, sem):
    c = lax.axis_index("c")
    s = lax.axis_index("s")
    w = s * NC + c
    @pl.loop(0, CH)
    def _fill0(i):
        rows[i] = jnp.zeros((H,), jnp.float32)

    @pl.loop(0, RPT // CH)
    def _zero(k):
        pltpu.sync_copy(rows, acc.at[pl.ds(s * RPT + k * CH, CH)])

    plsc.subcore_barrier()
    base = w * (PER_W // CH)
    @pl.loop(0, OUTER)
    def _outer(o):
        pltpu.sync_copy(src_hbm.at[pl.ds(base + o * IB, IB)], sidx)
        pltpu.sync_copy(dst_hbm.at[pl.ds(base + o * IB, IB)], didx)
        @pl.loop(0, IB)
        def _inner(j):
            pltpu.async_copy(u_hbm.at[sidx.at[j]], rows, sem).wait()
            pltpu.sync_copy(rows, acc.at[didx.at[j]], add=True)

    plsc.subcore_barrier()
    pltpu.sync_copy(acc.at[pl.ds(s * RPT, RPT)], out_hbm.at[c, pl.ds(s * RPT, RPT)])


def _sc_prop():
    return pl.kernel(
        _sc_prop_body,
        out_type=jax.ShapeDtypeStruct((NC, NP, H), jnp.float32),
        mesh=_mesh(),
        compiler_params=pltpu.CompilerParams(use_tc_tiling_on_sc=False),
        scratch_types=[
            pltpu.VMEM((IB, CH), jnp.int32),
            pltpu.VMEM((IB, CH), jnp.int32),
            pltpu.VMEM((CH, H), jnp.float32),
            pltpu.VMEM_SHARED((NP, H), jnp.float32),
            pltpu.SemaphoreType.DMA,
        ],
    )


def _sc_pool_body(a_hbm, b_hbm, out_hbm, bidx, rows, acc, sem):
    del sem
    c = lax.axis_index("c")
    s = lax.axis_index("s")
    w = s * NC + c
    @pl.loop(0, CH)
    def _fill0(i):
        rows[i] = jnp.zeros((H,), jnp.float32)

    pltpu.sync_copy(rows.at[pl.ds(0, 16)], acc.at[pl.ds(s * 16, 16)])
    plsc.subcore_barrier()
    nchunks = NP // NW // CH
    pltpu.sync_copy(b_hbm.at[pl.ds(w * nchunks, nchunks)], bidx)
    node0 = w * (NP // NW)
    @pl.loop(0, nchunks)
    def _run(j):
        pltpu.sync_copy(a_hbm.at[pl.ds(node0 + j * CH, CH)], rows)
        pltpu.sync_copy(rows, acc.at[bidx.at[j]], add=True)

    plsc.subcore_barrier()
    pltpu.sync_copy(acc.at[pl.ds(s * 8, 8)], out_hbm.at[c, pl.ds(s * 8, 8)])


def _sc_pool():
    return pl.kernel(
        _sc_pool_body,
        out_type=jax.ShapeDtypeStruct((NC, G, H), jnp.float32),
        mesh=_mesh(),
        compiler_params=pltpu.CompilerParams(use_tc_tiling_on_sc=False),
        scratch_types=[
            pltpu.VMEM((NP // NW // CH, CH), jnp.int32),
            pltpu.VMEM((CH, H), jnp.float32),
            pltpu.VMEM_SHARED((2 * G, H), jnp.float32),
            pltpu.SemaphoreType.DMA,
        ],
    )


def _tc_first_body(xp_r, d0_r, d1_r, w_r, u_r, dinv_r):
    dinv = lax.rsqrt(d0_r[...] + d1_r[...] + 1.0)
    u_r[...] = jnp.dot(xp_r[...], w_r[...], preferred_element_type=jnp.float32) * dinv
    dinv_r[...] = dinv


def _tc_first(xp, d0, d1, w1):
    return pl.pallas_call(
        _tc_first_body,
        grid=(NP // BM,),
        in_specs=[
            pl.BlockSpec((BM, F), lambda i: (i, 0)),
            pl.BlockSpec((BM, 1), lambda i: (i, 0)),
            pl.BlockSpec((BM, 1), lambda i: (i, 0)),
            pl.BlockSpec((F, H), lambda i: (0, 0)),
        ],
        out_specs=[
            pl.BlockSpec((BM, H), lambda i: (i, 0)),
            pl.BlockSpec((BM, 1), lambda i: (i, 0)),
        ],
        out_shape=[
            jax.ShapeDtypeStruct((NP, H), jnp.float32),
            jax.ShapeDtypeStruct((NP, 1), jnp.float32),
        ],
    )(xp, d0, d1, w1)


def _tc_mid_body(s0_r, s1_r, u_r, dinv_r, b_r, w_r, o_r):
    i = pl.program_id(0)
    dv = dinv_r[...]
    a = jnp.maximum(dv * (s0_r[...] + s1_r[...] + u_r[...]) + b_r[...], 0.0)
    row = lax.broadcasted_iota(jnp.int32, (BM, 1), 0) + i * BM
    msk = (row < N_NODES).astype(jnp.float32)
    o_r[...] = jnp.dot(a, w_r[...], preferred_element_type=jnp.float32) * dv * msk


def _tc_mid(s, u, dinv, b, w):
    return pl.pallas_call(
        _tc_mid_body,
        grid=(NP // BM,),
        in_specs=[
            pl.BlockSpec((BM, H), lambda i: (i, 0)),
            pl.BlockSpec((BM, H), lambda i: (i, 0)),
            pl.BlockSpec((BM, H), lambda i: (i, 0)),
            pl.BlockSpec((BM, 1), lambda i: (i, 0)),
            pl.BlockSpec((1, H), lambda i: (0, 0)),
            pl.BlockSpec((H, H), lambda i: (0, 0)),
        ],
        out_specs=pl.BlockSpec((BM, H), lambda i: (i, 0)),
        out_shape=jax.ShapeDtypeStruct((NP, H), jnp.float32),
    )(s[0], s[1], u, dinv, b[None], w)


def _tc_last_body(s0_r, s1_r, u_r, dinv_r, b_r, o_r):
    dv = dinv_r[...]
    o_r[...] = jnp.maximum(dv * (s0_r[...] + s1_r[...] + u_r[...]) + b_r[...], 0.0)


def _tc_last(s, u, dinv, b):
    return pl.pallas_call(
        _tc_last_body,
        grid=(NP // BM,),
        in_specs=[
            pl.BlockSpec((BM, H), lambda i: (i, 0)),
            pl.BlockSpec((BM, H), lambda i: (i, 0)),
            pl.BlockSpec((BM, H), lambda i: (i, 0)),
            pl.BlockSpec((BM, 1), lambda i: (i, 0)),
            pl.BlockSpec((1, H), lambda i: (0, 0)),
        ],
        out_specs=pl.BlockSpec((BM, H), lambda i: (i, 0)),
        out_shape=jax.ShapeDtypeStruct((NP, H), jnp.float32),
    )(s[0], s[1], u, dinv, b[None])


def _tc_head_body(p0_r, p1_r, y_r, w1a_r, w1b_r, b1_r, w2_r, b2_r, w3_r, b3_r, o_r):
    p = p0_r[...] + p1_r[...]
    z = jnp.dot(p, w1a_r[...], preferred_element_type=jnp.float32)
    z = z + jnp.dot(y_r[...], w1b_r[...], preferred_element_type=jnp.float32)
    z = jnp.maximum(z + b1_r[...], 0.0)
    z = jnp.maximum(jnp.dot(z, w2_r[...], preferred_element_type=jnp.float32) + b2_r[...], 0.0)
    o_r[...] = jnp.dot(z, w3_r[...], preferred_element_type=jnp.float32) + b3_r[...]


def _tc_head(pool, y, fc1_W, fc1_b, fc2_W, fc2_b, fc3_W, fc3_b):
    return pl.pallas_call(
        _tc_head_body,
        out_shape=jax.ShapeDtypeStruct((G, 1), jnp.float32),
    )(pool[0], pool[1], y, fc1_W[:H], fc1_W[H:], fc1_b[None], fc2_W,
      fc2_b[None], fc3_W, fc3_b[None])


def kernel(x, edge_index, y, batch, W1, b1, W2, b2, W3, b3,
           fc1_W, fc1_b, fc2_W, fc2_b, fc3_W, fc3_b):
    pad = jnp.full((EPAD - N_EDGES,), N_NODES, jnp.int32)
    src2 = jnp.concatenate([edge_index[0], pad]).reshape(-1, CH)
    dst2 = jnp.concatenate([edge_index[1], pad]).reshape(-1, CH)
    xp = jnp.pad(x, ((0, NP - N_NODES), (0, 0)))
    batch2 = jnp.pad(batch, (0, NP - N_NODES), constant_values=G).reshape(-1, CH)

    deg = _sc_deg()(dst2)
    d0 = deg[0][:, None]
    d1 = deg[1][:, None]
    u1, dinv = _tc_first(xp, d0, d1, W1)
    s1 = _sc_prop()(u1, src2, dst2)
    u2 = _tc_mid(s1, u1, dinv, b1, W2)
    s2 = _sc_prop()(u2, src2, dst2)
    u3 = _tc_mid(s2, u2, dinv, b2, W3)
    s3 = _sc_prop()(u3, src2, dst2)
    a3 = _tc_last(s3, u3, dinv, b3)
    pool = _sc_pool()(a3, batch2)
    return _tc_head(pool, y, fc1_W, fc1_b, fc2_W, fc2_b, fc3_W, fc3_b)


# speedup vs baseline: 36.3223x; 1.2829x over previous
import jax
import jax.numpy as jnp
from jax import lax
from jax.experimental import pallas as pl
from jax.experimental.pallas import tpu as pltpu
from jax.experimental.pallas import tpu_sc as plsc

N_NODES = 100000
N_EDGES = 3200000
F = 4
H = 16
G = 128
NC, NS = 2, 16
NW = NC * NS
CH = 128
IB = 16
OUTER = 49
PER_W = CH * IB * OUTER
EPAD = PER_W * NW
NP = 102400
RPT = NP // NS
BM = 4096


def _mesh():
    return plsc.VectorSubcoreMesh(core_axis_name="c", subcore_axis_name="s")


def _sc_deg_body(dst_hbm, out_hbm, didx, ones, acc, sem):
    c = lax.axis_index("c")
    s = lax.axis_index("s")
    w = s * NC + c
    @pl.loop(0, 8)
    def _fill0(i):
        ones[pl.ds(i * 16, 16)] = jnp.zeros((16,), jnp.float32)

    @pl.loop(0, RPT // CH)
    def _zero(k):
        pltpu.sync_copy(ones, acc.at[pl.ds(s * RPT + k * CH, CH)])

    @pl.loop(0, 8)
    def _fill1(i):
        ones[pl.ds(i * 16, 16)] = jnp.ones((16,), jnp.float32)

    plsc.subcore_barrier()
    base = w * (PER_W // CH)
    pltpu.sync_copy(dst_hbm.at[pl.ds(base, PER_W // CH)], didx)
    LAG = 8

    @pl.loop(0, PER_W // CH)
    def _scat(j):
        pltpu.async_copy(ones, acc.at[didx.at[j]], sem, add=True)
        @pl.when(j >= LAG)
        def _drain():
            pltpu.make_async_copy(ones, acc.at[didx.at[j]], sem).wait()

    for _ in range(LAG):
        pltpu.make_async_copy(ones, acc.at[didx.at[0]], sem).wait()

    plsc.subcore_barrier()
    pltpu.sync_copy(acc.at[pl.ds(s * RPT, RPT)], out_hbm.at[c, pl.ds(s * RPT, RPT)])


def _sc_deg():
    return pl.kernel(
        _sc_deg_body,
        out_type=jax.ShapeDtypeStruct((NC, NP), jnp.float32),
        mesh=_mesh(),
        compiler_params=pltpu.CompilerParams(use_tc_tiling_on_sc=False),
        scratch_types=[
            pltpu.VMEM((PER_W // CH, CH), jnp.int32),
            pltpu.VMEM((CH,), jnp.float32),
            pltpu.VMEM_SHARED((NP,), jnp.float32),
            pltpu.SemaphoreType.DMA,
        ],
    )


ROWS_W = PER_W // CH
BLK = 56
NBLK = ROWS_W // BLK


def _sc_prop_body(u_hbm, edges_hbm, out_hbm, idxv,
                  b0, b1, b2, b3, acc, gsem, ssem):
    c = lax.axis_index("c")
    s = lax.axis_index("s")
    w = s * NC + c
    bufs = (b0, b1, b2, b3)
    @pl.loop(0, CH)
    def _fill0(i):
        b0[i] = jnp.zeros((H,), jnp.float32)

    @pl.loop(0, RPT // CH)
    def _zero(k):
        pltpu.sync_copy(b0, acc.at[pl.ds(s * RPT + k * CH, CH)])

    plsc.subcore_barrier()

    def gissue(jj, buf):
        pltpu.async_copy(u_hbm.at[idxv.at[jj, 0]], buf, gsem)

    def gwait(jj, buf):
        pltpu.make_async_copy(u_hbm.at[idxv.at[jj, 0]], buf, gsem).wait()

    def sissue(jj, buf):
        pltpu.async_copy(buf, acc.at[idxv.at[jj, 1]], ssem, add=True)

    def swait(jj, buf):
        pltpu.make_async_copy(buf, acc.at[idxv.at[jj, 1]], ssem).wait()

    base = w * ROWS_W

    @pl.loop(0, NBLK)
    def _block(k):
        pltpu.sync_copy(edges_hbm.at[pl.ds(base + k * BLK, BLK)], idxv)
        gissue(0, bufs[0])
        gissue(1, bufs[1])
        gwait(0, bufs[0])
        sissue(0, bufs[0])
        gissue(2, bufs[2])
        gwait(1, bufs[1])
        sissue(1, bufs[1])
        gissue(3, bufs[3])

        @pl.loop(0, (BLK - 4) // 4)
        def _steady(q):
            j0 = 2 + q * 4
            for ro in range(4):
                jj = j0 + ro
                r = (2 + ro) % 4
                gwait(jj, bufs[r])
                sissue(jj, bufs[r])
                swait(jj - 2, bufs[(r + 2) % 4])
                gissue(jj + 2, bufs[(r + 2) % 4])

        for jj in (BLK - 2, BLK - 1):
            r = jj % 4
            gwait(jj, bufs[r])
            sissue(jj, bufs[r])
            swait(jj - 2, bufs[(r + 2) % 4])
        swait(BLK - 2, bufs[(BLK - 2) % 4])
        swait(BLK - 1, bufs[(BLK - 1) % 4])

    plsc.subcore_barrier()
    pltpu.sync_copy(acc.at[pl.ds(s * RPT, RPT)], out_hbm.at[c, pl.ds(s * RPT, RPT)])


def _sc_prop():
    return pl.kernel(
        _sc_prop_body,
        out_type=jax.ShapeDtypeStruct((NC, NP, H), jnp.float32),
        mesh=_mesh(),
        compiler_params=pltpu.CompilerParams(use_tc_tiling_on_sc=False),
        scratch_types=[
            pltpu.VMEM((BLK, 2, CH), jnp.int32),
            pltpu.VMEM((CH, H), jnp.float32),
            pltpu.VMEM((CH, H), jnp.float32),
            pltpu.VMEM((CH, H), jnp.float32),
            pltpu.VMEM((CH, H), jnp.float32),
            pltpu.VMEM_SHARED((NP, H), jnp.float32),
            pltpu.SemaphoreType.DMA,
            pltpu.SemaphoreType.DMA,
        ],
    )


def _sc_pool_body(a_hbm, b_hbm, out_hbm, bidx, rows, acc, sem):
    del sem
    c = lax.axis_index("c")
    s = lax.axis_index("s")
    w = s * NC + c
    @pl.loop(0, CH)
    def _fill0(i):
        rows[i] = jnp.zeros((H,), jnp.float32)

    pltpu.sync_copy(rows.at[pl.ds(0, 16)], acc.at[pl.ds(s * 16, 16)])
    plsc.subcore_barrier()
    nchunks = NP // NW // CH
    pltpu.sync_copy(b_hbm.at[pl.ds(w * nchunks, nchunks)], bidx)
    node0 = w * (NP // NW)
    @pl.loop(0, nchunks)
    def _run(j):
        pltpu.sync_copy(a_hbm.at[pl.ds(node0 + j * CH, CH)], rows)
        pltpu.sync_copy(rows, acc.at[bidx.at[j]], add=True)

    plsc.subcore_barrier()
    pltpu.sync_copy(acc.at[pl.ds(s * 8, 8)], out_hbm.at[c, pl.ds(s * 8, 8)])


def _sc_pool():
    return pl.kernel(
        _sc_pool_body,
        out_type=jax.ShapeDtypeStruct((NC, G, H), jnp.float32),
        mesh=_mesh(),
        compiler_params=pltpu.CompilerParams(use_tc_tiling_on_sc=False),
        scratch_types=[
            pltpu.VMEM((NP // NW // CH, CH), jnp.int32),
            pltpu.VMEM((CH, H), jnp.float32),
            pltpu.VMEM_SHARED((2 * G, H), jnp.float32),
            pltpu.SemaphoreType.DMA,
        ],
    )


def _tc_first_body(xp_r, d0_r, d1_r, w_r, u_r, dinv_r):
    dinv = lax.rsqrt(d0_r[...] + d1_r[...] + 1.0)
    u_r[...] = jnp.dot(xp_r[...], w_r[...], preferred_element_type=jnp.float32) * dinv
    dinv_r[...] = dinv


def _tc_first(xp, d0, d1, w1):
    return pl.pallas_call(
        _tc_first_body,
        grid=(NP // BM,),
        in_specs=[
            pl.BlockSpec((BM, F), lambda i: (i, 0)),
            pl.BlockSpec((BM, 1), lambda i: (i, 0)),
            pl.BlockSpec((BM, 1), lambda i: (i, 0)),
            pl.BlockSpec((F, H), lambda i: (0, 0)),
        ],
        out_specs=[
            pl.BlockSpec((BM, H), lambda i: (i, 0)),
            pl.BlockSpec((BM, 1), lambda i: (i, 0)),
        ],
        out_shape=[
            jax.ShapeDtypeStruct((NP, H), jnp.float32),
            jax.ShapeDtypeStruct((NP, 1), jnp.float32),
        ],
    )(xp, d0, d1, w1)


def _tc_mid_body(s0_r, s1_r, u_r, dinv_r, b_r, w_r, o_r):
    i = pl.program_id(0)
    dv = dinv_r[...]
    a = jnp.maximum(dv * (s0_r[...] + s1_r[...] + u_r[...]) + b_r[...], 0.0)
    row = lax.broadcasted_iota(jnp.int32, (BM, 1), 0) + i * BM
    msk = (row < N_NODES).astype(jnp.float32)
    o_r[...] = jnp.dot(a, w_r[...], preferred_element_type=jnp.float32) * dv * msk


def _tc_mid(s, u, dinv, b, w):
    return pl.pallas_call(
        _tc_mid_body,
        grid=(NP // BM,),
        in_specs=[
            pl.BlockSpec((BM, H), lambda i: (i, 0)),
            pl.BlockSpec((BM, H), lambda i: (i, 0)),
            pl.BlockSpec((BM, H), lambda i: (i, 0)),
            pl.BlockSpec((BM, 1), lambda i: (i, 0)),
            pl.BlockSpec((1, H), lambda i: (0, 0)),
            pl.BlockSpec((H, H), lambda i: (0, 0)),
        ],
        out_specs=pl.BlockSpec((BM, H), lambda i: (i, 0)),
        out_shape=jax.ShapeDtypeStruct((NP, H), jnp.float32),
    )(s[0], s[1], u, dinv, b[None], w)


def _tc_last_body(s0_r, s1_r, u_r, dinv_r, b_r, o_r):
    dv = dinv_r[...]
    o_r[...] = jnp.maximum(dv * (s0_r[...] + s1_r[...] + u_r[...]) + b_r[...], 0.0)


def _tc_last(s, u, dinv, b):
    return pl.pallas_call(
        _tc_last_body,
        grid=(NP // BM,),
        in_specs=[
            pl.BlockSpec((BM, H), lambda i: (i, 0)),
            pl.BlockSpec((BM, H), lambda i: (i, 0)),
            pl.BlockSpec((BM, H), lambda i: (i, 0)),
            pl.BlockSpec((BM, 1), lambda i: (i, 0)),
            pl.BlockSpec((1, H), lambda i: (0, 0)),
        ],
        out_specs=pl.BlockSpec((BM, H), lambda i: (i, 0)),
        out_shape=jax.ShapeDtypeStruct((NP, H), jnp.float32),
    )(s[0], s[1], u, dinv, b[None])


def _tc_head_body(p0_r, p1_r, y_r, w1a_r, w1b_r, b1_r, w2_r, b2_r, w3_r, b3_r, o_r):
    p = p0_r[...] + p1_r[...]
    z = jnp.dot(p, w1a_r[...], preferred_element_type=jnp.float32)
    z = z + jnp.dot(y_r[...], w1b_r[...], preferred_element_type=jnp.float32)
    z = jnp.maximum(z + b1_r[...], 0.0)
    z = jnp.maximum(jnp.dot(z, w2_r[...], preferred_element_type=jnp.float32) + b2_r[...], 0.0)
    o_r[...] = jnp.dot(z, w3_r[...], preferred_element_type=jnp.float32) + b3_r[...]


def _tc_head(pool, y, fc1_W, fc1_b, fc2_W, fc2_b, fc3_W, fc3_b):
    return pl.pallas_call(
        _tc_head_body,
        out_shape=jax.ShapeDtypeStruct((G, 1), jnp.float32),
    )(pool[0], pool[1], y, fc1_W[:H], fc1_W[H:], fc1_b[None], fc2_W,
      fc2_b[None], fc3_W, fc3_b[None])


def kernel(x, edge_index, y, batch, W1, b1, W2, b2, W3, b3,
           fc1_W, fc1_b, fc2_W, fc2_b, fc3_W, fc3_b):
    pad = jnp.full((EPAD - N_EDGES,), N_NODES, jnp.int32)
    src2 = jnp.concatenate([edge_index[0], pad]).reshape(-1, CH)
    dst2 = jnp.concatenate([edge_index[1], pad]).reshape(-1, CH)
    edges2 = jnp.stack([src2, dst2], axis=1)
    xp = jnp.pad(x, ((0, NP - N_NODES), (0, 0)))
    batch2 = jnp.pad(batch, (0, NP - N_NODES), constant_values=G).reshape(-1, CH)

    deg = _sc_deg()(dst2)
    d0 = deg[0][:, None]
    d1 = deg[1][:, None]
    u1, dinv = _tc_first(xp, d0, d1, W1)
    s1 = _sc_prop()(u1, edges2)
    u2 = _tc_mid(s1, u1, dinv, b1, W2)
    s2 = _sc_prop()(u2, edges2)
    u3 = _tc_mid(s2, u2, dinv, b2, W3)
    s3 = _sc_prop()(u3, edges2)
    a3 = _tc_last(s3, u3, dinv, b3)
    pool = _sc_pool()(a3, batch2)
    return _tc_head(pool, y, fc1_W, fc1_b, fc2_W, fc2_b, fc3_W, fc3_b)


# speedup vs baseline: 45.2468x; 1.2457x over previous
import jax
import jax.numpy as jnp
from jax import lax
from jax.experimental import pallas as pl
from jax.experimental.pallas import tpu as pltpu
from jax.experimental.pallas import tpu_sc as plsc

N_NODES = 100000
N_EDGES = 3200000
F = 4
H = 16
G = 128
NC, NS = 2, 16
NW = NC * NS
CH = 128
IB = 16
OUTER = 49
PER_W = CH * IB * OUTER
EPAD = PER_W * NW
NP = 102400
RPT = NP // NS
BM = 4096


def _mesh():
    return plsc.VectorSubcoreMesh(core_axis_name="c", subcore_axis_name="s")


def _sc_deg_body(dst_hbm, out_hbm, didx, ones, acc, sem):
    c = lax.axis_index("c")
    s = lax.axis_index("s")
    w = s * NC + c
    @pl.loop(0, 8)
    def _fill0(i):
        ones[pl.ds(i * 16, 16)] = jnp.zeros((16,), jnp.float32)

    @pl.loop(0, RPT // CH)
    def _zero(k):
        pltpu.sync_copy(ones, acc.at[pl.ds(s * RPT + k * CH, CH)])

    @pl.loop(0, 8)
    def _fill1(i):
        ones[pl.ds(i * 16, 16)] = jnp.ones((16,), jnp.float32)

    plsc.subcore_barrier()
    base = w * (PER_W // CH)
    pltpu.sync_copy(dst_hbm.at[pl.ds(base, PER_W // CH)], didx)
    LAG = 8

    @pl.loop(0, PER_W // CH)
    def _scat(j):
        pltpu.async_copy(ones, acc.at[didx.at[j]], sem, add=True)
        @pl.when(j >= LAG)
        def _drain():
            pltpu.make_async_copy(ones, acc.at[didx.at[j]], sem).wait()

    for _ in range(LAG):
        pltpu.make_async_copy(ones, acc.at[didx.at[0]], sem).wait()

    plsc.subcore_barrier()
    pltpu.sync_copy(acc.at[pl.ds(s * RPT, RPT)], out_hbm.at[c, pl.ds(s * RPT, RPT)])


def _sc_deg():
    return pl.kernel(
        _sc_deg_body,
        out_type=jax.ShapeDtypeStruct((NC, NP), jnp.float32),
        mesh=_mesh(),
        compiler_params=pltpu.CompilerParams(use_tc_tiling_on_sc=False),
        scratch_types=[
            pltpu.VMEM((PER_W // CH, CH), jnp.int32),
            pltpu.VMEM((CH,), jnp.float32),
            pltpu.VMEM_SHARED((NP,), jnp.float32),
            pltpu.SemaphoreType.DMA,
        ],
    )


ROWS_W = PER_W // CH
BLK = 28
NBLK = ROWS_W // BLK
NPAIR = BLK // 2


def _sc_prop_body(u_hbm, edges_hbm, out_hbm, idxv,
                  b0, b1, b2, b3, acc, gsem, ssem):
    c = lax.axis_index("c")
    s = lax.axis_index("s")
    w = s * NC + c
    bufs = (b0, b1, b2, b3)
    @pl.loop(0, 2 * CH)
    def _fill0(i):
        b0[i] = jnp.zeros((H,), jnp.float32)

    @pl.loop(0, RPT // (2 * CH))
    def _zero(k):
        pltpu.sync_copy(b0, acc.at[pl.ds(s * RPT + k * 2 * CH, 2 * CH)])

    plsc.subcore_barrier()

    def pg(p, r):
        pltpu.async_copy(u_hbm.at[idxv.at[2 * p, 0]], bufs[r].at[pl.ds(0, CH)], gsem)
        pltpu.async_copy(u_hbm.at[idxv.at[2 * p + 1, 0]], bufs[r].at[pl.ds(CH, CH)], gsem)

    def pgwait(p, r):
        pltpu.make_async_copy(u_hbm.at[idxv.at[2 * p, 0]], bufs[r].at[pl.ds(0, CH)], gsem).wait()
        pltpu.make_async_copy(u_hbm.at[idxv.at[2 * p + 1, 0]], bufs[r].at[pl.ds(CH, CH)], gsem).wait()

    def ps(p, r):
        pltpu.async_copy(bufs[r].at[pl.ds(0, CH)], acc.at[idxv.at[2 * p, 1]], ssem, add=True)
        pltpu.async_copy(bufs[r].at[pl.ds(CH, CH)], acc.at[idxv.at[2 * p + 1, 1]], ssem, add=True)

    def pswait(p, r):
        pltpu.make_async_copy(bufs[r].at[pl.ds(0, CH)], acc.at[idxv.at[2 * p, 1]], ssem).wait()
        pltpu.make_async_copy(bufs[r].at[pl.ds(CH, CH)], acc.at[idxv.at[2 * p + 1, 1]], ssem).wait()

    base = w * ROWS_W

    @pl.loop(0, NBLK)
    def _block(k):
        pltpu.sync_copy(edges_hbm.at[pl.ds(base + k * BLK, BLK)], idxv)
        pg(0, 0)
        pg(1, 1)
        for p in (0, 1):
            pgwait(p, p)
            ps(p, p)
            pg(p + 2, p + 2)

        def steady(p, r):
            pgwait(p, r)
            ps(p, r)
            pswait(p - 2, (r + 2) % 4)
            pg(p + 2, (r + 2) % 4)

        @pl.loop(0, 2)
        def _steady(q):
            p0 = 2 + q * 4
            for ro in range(4):
                steady(p0 + ro, (2 + ro) % 4)

        for p in (10, 11):
            steady(p, p % 4)
        for p in (12, 13):
            r = p % 4
            pgwait(p, r)
            ps(p, r)
            pswait(p - 2, (r + 2) % 4)
        pswait(12, 0)
        pswait(13, 1)

    plsc.subcore_barrier()
    pltpu.sync_copy(acc.at[pl.ds(s * RPT, RPT)], out_hbm.at[c, pl.ds(s * RPT, RPT)])


def _sc_prop():
    return pl.kernel(
        _sc_prop_body,
        out_type=jax.ShapeDtypeStruct((NC, NP, H), jnp.float32),
        mesh=_mesh(),
        compiler_params=pltpu.CompilerParams(use_tc_tiling_on_sc=False),
        scratch_types=[
            pltpu.VMEM((BLK, 2, CH), jnp.int32),
            pltpu.VMEM((2 * CH, H), jnp.float32),
            pltpu.VMEM((2 * CH, H), jnp.float32),
            pltpu.VMEM((2 * CH, H), jnp.float32),
            pltpu.VMEM((2 * CH, H), jnp.float32),
            pltpu.VMEM_SHARED((NP, H), jnp.float32),
            pltpu.SemaphoreType.DMA,
            pltpu.SemaphoreType.DMA,
        ],
    )


def _sc_pool_body(a_hbm, b_hbm, out_hbm, bidx, rows, acc, sem):
    del sem
    c = lax.axis_index("c")
    s = lax.axis_index("s")
    w = s * NC + c
    @pl.loop(0, CH)
    def _fill0(i):
        rows[i] = jnp.zeros((H,), jnp.float32)

    pltpu.sync_copy(rows.at[pl.ds(0, 16)], acc.at[pl.ds(s * 16, 16)])
    plsc.subcore_barrier()
    nchunks = NP // NW // CH
    pltpu.sync_copy(b_hbm.at[pl.ds(w * nchunks, nchunks)], bidx)
    node0 = w * (NP // NW)
    @pl.loop(0, nchunks)
    def _run(j):
        pltpu.sync_copy(a_hbm.at[pl.ds(node0 + j * CH, CH)], rows)
        pltpu.sync_copy(rows, acc.at[bidx.at[j]], add=True)

    plsc.subcore_barrier()
    pltpu.sync_copy(acc.at[pl.ds(s * 8, 8)], out_hbm.at[c, pl.ds(s * 8, 8)])


def _sc_pool():
    return pl.kernel(
        _sc_pool_body,
        out_type=jax.ShapeDtypeStruct((NC, G, H), jnp.float32),
        mesh=_mesh(),
        compiler_params=pltpu.CompilerParams(use_tc_tiling_on_sc=False),
        scratch_types=[
            pltpu.VMEM((NP // NW // CH, CH), jnp.int32),
            pltpu.VMEM((CH, H), jnp.float32),
            pltpu.VMEM_SHARED((2 * G, H), jnp.float32),
            pltpu.SemaphoreType.DMA,
        ],
    )


def _tc_first_body(xp_r, d0_r, d1_r, w_r, u_r, dinv_r):
    dinv = lax.rsqrt(d0_r[...] + d1_r[...] + 1.0)
    u_r[...] = jnp.dot(xp_r[...], w_r[...], preferred_element_type=jnp.float32) * dinv
    dinv_r[...] = dinv


def _tc_first(xp, d0, d1, w1):
    return pl.pallas_call(
        _tc_first_body,
        grid=(NP // BM,),
        in_specs=[
            pl.BlockSpec((BM, F), lambda i: (i, 0)),
            pl.BlockSpec((BM, 1), lambda i: (i, 0)),
            pl.BlockSpec((BM, 1), lambda i: (i, 0)),
            pl.BlockSpec((F, H), lambda i: (0, 0)),
        ],
        out_specs=[
            pl.BlockSpec((BM, H), lambda i: (i, 0)),
            pl.BlockSpec((BM, 1), lambda i: (i, 0)),
        ],
        out_shape=[
            jax.ShapeDtypeStruct((NP, H), jnp.float32),
            jax.ShapeDtypeStruct((NP, 1), jnp.float32),
        ],
    )(xp, d0, d1, w1)


def _tc_mid_body(s0_r, s1_r, u_r, dinv_r, b_r, w_r, o_r):
    i = pl.program_id(0)
    dv = dinv_r[...]
    a = jnp.maximum(dv * (s0_r[...] + s1_r[...] + u_r[...]) + b_r[...], 0.0)
    row = lax.broadcasted_iota(jnp.int32, (BM, 1), 0) + i * BM
    msk = (row < N_NODES).astype(jnp.float32)
    o_r[...] = jnp.dot(a, w_r[...], preferred_element_type=jnp.float32) * dv * msk


def _tc_mid(s, u, dinv, b, w):
    return pl.pallas_call(
        _tc_mid_body,
        grid=(NP // BM,),
        in_specs=[
            pl.BlockSpec((BM, H), lambda i: (i, 0)),
            pl.BlockSpec((BM, H), lambda i: (i, 0)),
            pl.BlockSpec((BM, H), lambda i: (i, 0)),
            pl.BlockSpec((BM, 1), lambda i: (i, 0)),
            pl.BlockSpec((1, H), lambda i: (0, 0)),
            pl.BlockSpec((H, H), lambda i: (0, 0)),
        ],
        out_specs=pl.BlockSpec((BM, H), lambda i: (i, 0)),
        out_shape=jax.ShapeDtypeStruct((NP, H), jnp.float32),
    )(s[0], s[1], u, dinv, b[None], w)


def _tc_last_body(s0_r, s1_r, u_r, dinv_r, b_r, o_r):
    dv = dinv_r[...]
    o_r[...] = jnp.maximum(dv * (s0_r[...] + s1_r[...] + u_r[...]) + b_r[...], 0.0)


def _tc_last(s, u, dinv, b):
    return pl.pallas_call(
        _tc_last_body,
        grid=(NP // BM,),
        in_specs=[
            pl.BlockSpec((BM, H), lambda i: (i, 0)),
            pl.BlockSpec((BM, H), lambda i: (i, 0)),
            pl.BlockSpec((BM, H), lambda i: (i, 0)),
            pl.BlockSpec((BM, 1), lambda i: (i, 0)),
            pl.BlockSpec((1, H), lambda i: (0, 0)),
        ],
        out_specs=pl.BlockSpec((BM, H), lambda i: (i, 0)),
        out_shape=jax.ShapeDtypeStruct((NP, H), jnp.float32),
    )(s[0], s[1], u, dinv, b[None])


def _tc_head_body(p0_r, p1_r, y_r, w1a_r, w1b_r, b1_r, w2_r, b2_r, w3_r, b3_r, o_r):
    p = p0_r[...] + p1_r[...]
    z = jnp.dot(p, w1a_r[...], preferred_element_type=jnp.float32)
    z = z + jnp.dot(y_r[...], w1b_r[...], preferred_element_type=jnp.float32)
    z = jnp.maximum(z + b1_r[...], 0.0)
    z = jnp.maximum(jnp.dot(z, w2_r[...], preferred_element_type=jnp.float32) + b2_r[...], 0.0)
    o_r[...] = jnp.dot(z, w3_r[...], preferred_element_type=jnp.float32) + b3_r[...]


def _tc_head(pool, y, fc1_W, fc1_b, fc2_W, fc2_b, fc3_W, fc3_b):
    return pl.pallas_call(
        _tc_head_body,
        out_shape=jax.ShapeDtypeStruct((G, 1), jnp.float32),
    )(pool[0], pool[1], y, fc1_W[:H], fc1_W[H:], fc1_b[None], fc2_W,
      fc2_b[None], fc3_W, fc3_b[None])


def kernel(x, edge_index, y, batch, W1, b1, W2, b2, W3, b3,
           fc1_W, fc1_b, fc2_W, fc2_b, fc3_W, fc3_b):
    pad = jnp.full((EPAD - N_EDGES,), N_NODES, jnp.int32)
    src2 = jnp.concatenate([edge_index[0], pad]).reshape(-1, CH)
    dst2 = jnp.concatenate([edge_index[1], pad]).reshape(-1, CH)
    edges2 = jnp.stack([src2, dst2], axis=1)
    xp = jnp.pad(x, ((0, NP - N_NODES), (0, 0)))
    batch2 = jnp.pad(batch, (0, NP - N_NODES), constant_values=G).reshape(-1, CH)

    deg = _sc_deg()(dst2)
    d0 = deg[0][:, None]
    d1 = deg[1][:, None]
    u1, dinv = _tc_first(xp, d0, d1, W1)
    s1 = _sc_prop()(u1, edges2)
    u2 = _tc_mid(s1, u1, dinv, b1, W2)
    s2 = _sc_prop()(u2, edges2)
    u3 = _tc_mid(s2, u2, dinv, b2, W3)
    s3 = _sc_prop()(u3, edges2)
    a3 = _tc_last(s3, u3, dinv, b3)
    pool = _sc_pool()(a3, batch2)
    return _tc_head(pool, y, fc1_W, fc1_b, fc2_W, fc2_b, fc3_W, fc3_b)


# speedup vs baseline: 46.0232x; 1.0172x over previous
import jax
import jax.numpy as jnp
from jax import lax
from jax.experimental import pallas as pl
from jax.experimental.pallas import tpu as pltpu
from jax.experimental.pallas import tpu_sc as plsc

N_NODES = 100000
N_EDGES = 3200000
F = 4
H = 16
G = 128
NC, NS = 2, 16
NW = NC * NS
CH = 128
IB = 16
OUTER = 49
PER_W = CH * IB * OUTER
EPAD = PER_W * NW
NP = 102400
RPT = NP // NS
BM = 4096


def _mesh():
    return plsc.VectorSubcoreMesh(core_axis_name="c", subcore_axis_name="s")


def _sc_deg_body(dst_hbm, out_hbm, didx, ones, acc, sem):
    c = lax.axis_index("c")
    s = lax.axis_index("s")
    w = s * NC + c
    @pl.loop(0, 8)
    def _fill0(i):
        ones[pl.ds(i * 16, 16)] = jnp.zeros((16,), jnp.float32)

    @pl.loop(0, RPT // CH)
    def _zero(k):
        pltpu.sync_copy(ones, acc.at[pl.ds(s * RPT + k * CH, CH)])

    @pl.loop(0, 8)
    def _fill1(i):
        ones[pl.ds(i * 16, 16)] = jnp.ones((16,), jnp.float32)

    plsc.subcore_barrier()
    base = w * (PER_W // CH)
    pltpu.sync_copy(dst_hbm.at[pl.ds(base, PER_W // CH)], didx)
    LAG = 8

    @pl.loop(0, PER_W // CH)
    def _scat(j):
        pltpu.async_copy(ones, acc.at[didx.at[j]], sem, add=True)
        @pl.when(j >= LAG)
        def _drain():
            pltpu.make_async_copy(ones, acc.at[didx.at[j]], sem).wait()

    for _ in range(LAG):
        pltpu.make_async_copy(ones, acc.at[didx.at[0]], sem).wait()

    plsc.subcore_barrier()
    pltpu.sync_copy(acc.at[pl.ds(s * RPT, RPT)], out_hbm.at[c, pl.ds(s * RPT, RPT)])


def _sc_deg():
    return pl.kernel(
        _sc_deg_body,
        out_type=jax.ShapeDtypeStruct((NC, NP), jnp.float32),
        mesh=_mesh(),
        compiler_params=pltpu.CompilerParams(use_tc_tiling_on_sc=False),
        scratch_types=[
            pltpu.VMEM((PER_W // CH, CH), jnp.int32),
            pltpu.VMEM((CH,), jnp.float32),
            pltpu.VMEM_SHARED((NP,), jnp.float32),
            pltpu.SemaphoreType.DMA,
        ],
    )


ROWS_W = PER_W // CH
BLK = 28
NBLK = ROWS_W // BLK
NPAIR = BLK // 2


def _sc_prop_body(u_hbm, edges_hbm, out_hbm, idxv,
                  b0, b1, b2, b3, acc, gsem, ssem):
    c = lax.axis_index("c")
    s = lax.axis_index("s")
    w = s * NC + c
    bufs = (b0, b1, b2, b3)
    @pl.loop(0, 2 * CH)
    def _fill0(i):
        b0[i] = jnp.zeros((H,), jnp.float32)

    @pl.loop(0, RPT // (2 * CH))
    def _zero(k):
        pltpu.sync_copy(b0, acc.at[pl.ds(s * RPT + k * 2 * CH, 2 * CH)])

    plsc.subcore_barrier()

    def pg(p, r):
        pltpu.async_copy(u_hbm.at[idxv.at[2 * p, 0]], bufs[r].at[pl.ds(0, CH)], gsem)
        pltpu.async_copy(u_hbm.at[idxv.at[2 * p + 1, 0]], bufs[r].at[pl.ds(CH, CH)], gsem)

    def pgwait(p, r):
        pltpu.make_async_copy(u_hbm.at[idxv.at[2 * p, 0]], bufs[r].at[pl.ds(0, CH)], gsem).wait()
        pltpu.make_async_copy(u_hbm.at[idxv.at[2 * p + 1, 0]], bufs[r].at[pl.ds(CH, CH)], gsem).wait()

    def ps(p, r):
        pltpu.async_copy(bufs[r].at[pl.ds(0, CH)], acc.at[idxv.at[2 * p, 1]], ssem, add=True)
        pltpu.async_copy(bufs[r].at[pl.ds(CH, CH)], acc.at[idxv.at[2 * p + 1, 1]], ssem, add=True)

    def pswait(p, r):
        pltpu.make_async_copy(bufs[r].at[pl.ds(0, CH)], acc.at[idxv.at[2 * p, 1]], ssem).wait()
        pltpu.make_async_copy(bufs[r].at[pl.ds(CH, CH)], acc.at[idxv.at[2 * p + 1, 1]], ssem).wait()

    base = w * ROWS_W

    @pl.loop(0, NBLK)
    def _block(k):
        pltpu.sync_copy(edges_hbm.at[pl.ds(base + k * BLK, BLK)], idxv)
        pg(0, 0)
        pg(1, 1)
        for p in (0, 1):
            pgwait(p, p)
            ps(p, p)
            pg(p + 2, p + 2)

        def steady(p, r):
            pgwait(p, r)
            ps(p, r)
            pswait(p - 2, (r + 2) % 4)
            pg(p + 2, (r + 2) % 4)

        @pl.loop(0, 2)
        def _steady(q):
            p0 = 2 + q * 4
            for ro in range(4):
                steady(p0 + ro, (2 + ro) % 4)

        for p in (10, 11):
            steady(p, p % 4)
        for p in (12, 13):
            r = p % 4
            pgwait(p, r)
            ps(p, r)
            pswait(p - 2, (r + 2) % 4)
        pswait(12, 0)
        pswait(13, 1)

    plsc.subcore_barrier()
    pltpu.sync_copy(acc.at[pl.ds(s * RPT, RPT)], out_hbm.at[c, pl.ds(s * RPT, RPT)])


def _sc_prop():
    return pl.kernel(
        _sc_prop_body,
        out_type=jax.ShapeDtypeStruct((NC, NP, H), jnp.float32),
        mesh=_mesh(),
        compiler_params=pltpu.CompilerParams(use_tc_tiling_on_sc=False),
        scratch_types=[
            pltpu.VMEM((BLK, 2, CH), jnp.int32),
            pltpu.VMEM((2 * CH, H), jnp.float32),
            pltpu.VMEM((2 * CH, H), jnp.float32),
            pltpu.VMEM((2 * CH, H), jnp.float32),
            pltpu.VMEM((2 * CH, H), jnp.float32),
            pltpu.VMEM_SHARED((NP, H), jnp.float32),
            pltpu.SemaphoreType.DMA,
            pltpu.SemaphoreType.DMA,
        ],
    )


def _tc_first_body(xp_r, d0_r, d1_r, w_r, u_r, dinv_r):
    dinv = lax.rsqrt(d0_r[...] + d1_r[...] + 1.0)
    u_r[...] = jnp.dot(xp_r[...], w_r[...], preferred_element_type=jnp.float32) * dinv
    dinv_r[...] = dinv


def _tc_first(xp, d0, d1, w1):
    return pl.pallas_call(
        _tc_first_body,
        grid=(NP // BM,),
        in_specs=[
            pl.BlockSpec((BM, F), lambda i: (i, 0)),
            pl.BlockSpec((BM, 1), lambda i: (i, 0)),
            pl.BlockSpec((BM, 1), lambda i: (i, 0)),
            pl.BlockSpec((F, H), lambda i: (0, 0)),
        ],
        out_specs=[
            pl.BlockSpec((BM, H), lambda i: (i, 0)),
            pl.BlockSpec((BM, 1), lambda i: (i, 0)),
        ],
        out_shape=[
            jax.ShapeDtypeStruct((NP, H), jnp.float32),
            jax.ShapeDtypeStruct((NP, 1), jnp.float32),
        ],
    )(xp, d0, d1, w1)


def _tc_mid_body(s0_r, s1_r, u_r, dinv_r, b_r, w_r, o_r):
    i = pl.program_id(0)
    dv = dinv_r[...]
    a = jnp.maximum(dv * (s0_r[...] + s1_r[...] + u_r[...]) + b_r[...], 0.0)
    row = lax.broadcasted_iota(jnp.int32, (BM, 1), 0) + i * BM
    msk = (row < N_NODES).astype(jnp.float32)
    o_r[...] = jnp.dot(a, w_r[...], preferred_element_type=jnp.float32) * dv * msk


def _tc_mid(s, u, dinv, b, w):
    return pl.pallas_call(
        _tc_mid_body,
        grid=(NP // BM,),
        in_specs=[
            pl.BlockSpec((BM, H), lambda i: (i, 0)),
            pl.BlockSpec((BM, H), lambda i: (i, 0)),
            pl.BlockSpec((BM, H), lambda i: (i, 0)),
            pl.BlockSpec((BM, 1), lambda i: (i, 0)),
            pl.BlockSpec((1, H), lambda i: (0, 0)),
            pl.BlockSpec((H, H), lambda i: (0, 0)),
        ],
        out_specs=pl.BlockSpec((BM, H), lambda i: (i, 0)),
        out_shape=jax.ShapeDtypeStruct((NP, H), jnp.float32),
    )(s[0], s[1], u, dinv, b[None], w)


def _tc_tail_body(s0_r, s1_r, u_r, dinv_r, b_r, bi_r, y_r, w1a_r, w1b_r,
                  b1_r, w2_r, b2_r, w3_r, b3_r, o_r, pacc):
    i = pl.program_id(0)
    dv = dinv_r[...]
    a = jnp.maximum(dv * (s0_r[...] + s1_r[...] + u_r[...]) + b_r[...], 0.0)
    oh = (bi_r[...] == lax.broadcasted_iota(jnp.int32, (BM, G), 1)).astype(jnp.float32)
    part = lax.dot_general(oh, a, (((0,), (0,)), ((), ())),
                           preferred_element_type=jnp.float32)

    @pl.when(i == 0)
    def _init():
        pacc[...] = part

    @pl.when(i > 0)
    def _accum():
        pacc[...] = pacc[...] + part

    @pl.when(i == NP // BM - 1)
    def _head():
        p = pacc[...]
        z = jnp.dot(p, w1a_r[...], preferred_element_type=jnp.float32)
        z = z + jnp.dot(y_r[...], w1b_r[...], preferred_element_type=jnp.float32)
        z = jnp.maximum(z + b1_r[...], 0.0)
        z = jnp.maximum(jnp.dot(z, w2_r[...], preferred_element_type=jnp.float32) + b2_r[...], 0.0)
        o_r[...] = jnp.dot(z, w3_r[...], preferred_element_type=jnp.float32) + b3_r[...]


def _tc_tail(s, u, dinv, b, batch_pi, y, fc1_W, fc1_b, fc2_W, fc2_b, fc3_W, fc3_b):
    return pl.pallas_call(
        _tc_tail_body,
        grid=(NP // BM,),
        in_specs=[
            pl.BlockSpec((BM, H), lambda i: (i, 0)),
            pl.BlockSpec((BM, H), lambda i: (i, 0)),
            pl.BlockSpec((BM, H), lambda i: (i, 0)),
            pl.BlockSpec((BM, 1), lambda i: (i, 0)),
            pl.BlockSpec((1, H), lambda i: (0, 0)),
            pl.BlockSpec((BM, 1), lambda i: (i, 0)),
            pl.BlockSpec((G, F), lambda i: (0, 0)),
            pl.BlockSpec((H, H), lambda i: (0, 0)),
            pl.BlockSpec((F, H), lambda i: (0, 0)),
            pl.BlockSpec((1, H), lambda i: (0, 0)),
            pl.BlockSpec((H, H), lambda i: (0, 0)),
            pl.BlockSpec((1, H), lambda i: (0, 0)),
            pl.BlockSpec((H, 1), lambda i: (0, 0)),
            pl.BlockSpec((1, 1), lambda i: (0, 0)),
        ],
        out_specs=pl.BlockSpec((G, 1), lambda i: (0, 0)),
        out_shape=jax.ShapeDtypeStruct((G, 1), jnp.float32),
        scratch_shapes=[pltpu.VMEM((G, H), jnp.float32)],
    )(s[0], s[1], u, dinv, b[None], batch_pi, y, fc1_W[:H], fc1_W[H:],
      fc1_b[None], fc2_W, fc2_b[None], fc3_W, fc3_b[None])


def kernel(x, edge_index, y, batch, W1, b1, W2, b2, W3, b3,
           fc1_W, fc1_b, fc2_W, fc2_b, fc3_W, fc3_b):
    pad = jnp.full((EPAD - N_EDGES,), N_NODES, jnp.int32)
    src2 = jnp.concatenate([edge_index[0], pad]).reshape(-1, CH)
    dst2 = jnp.concatenate([edge_index[1], pad]).reshape(-1, CH)
    edges2 = jnp.stack([src2, dst2], axis=1)
    xp = jnp.pad(x, ((0, NP - N_NODES), (0, 0)))
    batch_pi = jnp.pad(batch, (0, NP - N_NODES), constant_values=G)[:, None]

    deg = _sc_deg()(dst2)
    d0 = deg[0][:, None]
    d1 = deg[1][:, None]
    u1, dinv = _tc_first(xp, d0, d1, W1)
    s1 = _sc_prop()(u1, edges2)
    u2 = _tc_mid(s1, u1, dinv, b1, W2)
    s2 = _sc_prop()(u2, edges2)
    u3 = _tc_mid(s2, u2, dinv, b2, W3)
    s3 = _sc_prop()(u3, edges2)
    return _tc_tail(s3, u3, dinv, b3, batch_pi, y,
                    fc1_W, fc1_b, fc2_W, fc2_b, fc3_W, fc3_b)


# speedup vs baseline: 62.8337x; 1.3653x over previous
import jax
import jax.numpy as jnp
from jax import lax
from jax.experimental import pallas as pl
from jax.experimental.pallas import tpu as pltpu
from jax.experimental.pallas import tpu_sc as plsc

N_NODES = 100000
N_EDGES = 3200000
F = 4
H = 16
G = 128
NC, NS = 2, 16
NW = NC * NS
CH = 128
IB = 16
OUTER = 49
PER_W = CH * IB * OUTER
EPAD = PER_W * NW
NP = 102400
RPT = NP // NS
BM = 4096


def _mesh():
    return plsc.VectorSubcoreMesh(core_axis_name="c", subcore_axis_name="s")


DBLK = 56


def _sc_deg_body(dst_hbm, out_hbm, didx, ones, acc, sem):
    c = lax.axis_index("c")
    s = lax.axis_index("s")
    w = s * NC + c
    @pl.loop(0, CH)
    def _fill0(i):
        ones[i] = jnp.zeros((H,), jnp.float32)

    @pl.loop(0, RPT // CH)
    def _zero(k):
        pltpu.sync_copy(ones, acc.at[pl.ds(s * RPT + k * CH, CH)])

    @pl.loop(0, CH)
    def _fill1(i):
        ones[i] = jnp.ones((H,), jnp.float32)

    plsc.subcore_barrier()
    base = w * (PER_W // CH)
    LAG = 4

    @pl.loop(0, PER_W // CH // DBLK)
    def _blk(k):
        pltpu.sync_copy(dst_hbm.at[pl.ds(base + k * DBLK, DBLK)], didx)

        @pl.loop(0, DBLK)
        def _scat(j):
            pltpu.async_copy(ones, acc.at[didx.at[j]], sem, add=True)
            @pl.when(j >= LAG)
            def _drain():
                pltpu.make_async_copy(ones, acc.at[didx.at[j]], sem).wait()

        for _ in range(LAG):
            pltpu.make_async_copy(ones, acc.at[didx.at[0]], sem).wait()

    plsc.subcore_barrier()
    pltpu.sync_copy(acc.at[pl.ds(s * RPT, RPT)], out_hbm.at[c, pl.ds(s * RPT, RPT)])


def _sc_deg():
    return pl.kernel(
        _sc_deg_body,
        out_type=jax.ShapeDtypeStruct((NC, NP, H), jnp.float32),
        mesh=_mesh(),
        compiler_params=pltpu.CompilerParams(use_tc_tiling_on_sc=False),
        scratch_types=[
            pltpu.VMEM((DBLK, CH), jnp.int32),
            pltpu.VMEM((CH, H), jnp.float32),
            pltpu.VMEM_SHARED((NP, H), jnp.float32),
            pltpu.SemaphoreType.DMA,
        ],
    )


ROWS_W = PER_W // CH
BLK = 28
NBLK = ROWS_W // BLK
NPAIR = BLK // 2


def _sc_prop_body(u_hbm, edges_hbm, out_hbm, idxv,
                  b0, b1, b2, b3, acc, gsem, ssem):
    c = lax.axis_index("c")
    s = lax.axis_index("s")
    w = s * NC + c
    bufs = (b0, b1, b2, b3)
    @pl.loop(0, 2 * CH)
    def _fill0(i):
        b0[i] = jnp.zeros((H,), jnp.float32)

    @pl.loop(0, RPT // (2 * CH))
    def _zero(k):
        pltpu.sync_copy(b0, acc.at[pl.ds(s * RPT + k * 2 * CH, 2 * CH)])

    plsc.subcore_barrier()

    def pg(p, r):
        pltpu.async_copy(u_hbm.at[idxv.at[2 * p, 0]], bufs[r].at[pl.ds(0, CH)], gsem)
        pltpu.async_copy(u_hbm.at[idxv.at[2 * p + 1, 0]], bufs[r].at[pl.ds(CH, CH)], gsem)

    def pgwait(p, r):
        pltpu.make_async_copy(u_hbm.at[idxv.at[2 * p, 0]], bufs[r].at[pl.ds(0, CH)], gsem).wait()
        pltpu.make_async_copy(u_hbm.at[idxv.at[2 * p + 1, 0]], bufs[r].at[pl.ds(CH, CH)], gsem).wait()

    def ps(p, r):
        pltpu.async_copy(bufs[r].at[pl.ds(0, CH)], acc.at[idxv.at[2 * p, 1]], ssem, add=True)
        pltpu.async_copy(bufs[r].at[pl.ds(CH, CH)], acc.at[idxv.at[2 * p + 1, 1]], ssem, add=True)

    def pswait(p, r):
        pltpu.make_async_copy(bufs[r].at[pl.ds(0, CH)], acc.at[idxv.at[2 * p, 1]], ssem).wait()
        pltpu.make_async_copy(bufs[r].at[pl.ds(CH, CH)], acc.at[idxv.at[2 * p + 1, 1]], ssem).wait()

    base = w * ROWS_W

    @pl.loop(0, NBLK)
    def _block(k):
        pltpu.sync_copy(edges_hbm.at[pl.ds(base + k * BLK, BLK)], idxv)
        pg(0, 0)
        pg(1, 1)
        for p in (0, 1):
            pgwait(p, p)
            ps(p, p)
            pg(p + 2, p + 2)

        def steady(p, r):
            pgwait(p, r)
            ps(p, r)
            pswait(p - 2, (r + 2) % 4)
            pg(p + 2, (r + 2) % 4)

        @pl.loop(0, 2)
        def _steady(q):
            p0 = 2 + q * 4
            for ro in range(4):
                steady(p0 + ro, (2 + ro) % 4)

        for p in (10, 11):
            steady(p, p % 4)
        for p in (12, 13):
            r = p % 4
            pgwait(p, r)
            ps(p, r)
            pswait(p - 2, (r + 2) % 4)
        pswait(12, 0)
        pswait(13, 1)

    plsc.subcore_barrier()
    pltpu.sync_copy(acc.at[pl.ds(s * RPT, RPT)], out_hbm.at[c, pl.ds(s * RPT, RPT)])


def _sc_prop():
    return pl.kernel(
        _sc_prop_body,
        out_type=jax.ShapeDtypeStruct((NC, NP, H), jnp.float32),
        mesh=_mesh(),
        compiler_params=pltpu.CompilerParams(use_tc_tiling_on_sc=False),
        scratch_types=[
            pltpu.VMEM((BLK, 2, CH), jnp.int32),
            pltpu.VMEM((2 * CH, H), jnp.float32),
            pltpu.VMEM((2 * CH, H), jnp.float32),
            pltpu.VMEM((2 * CH, H), jnp.float32),
            pltpu.VMEM((2 * CH, H), jnp.float32),
            pltpu.VMEM_SHARED((NP, H), jnp.float32),
            pltpu.SemaphoreType.DMA,
            pltpu.SemaphoreType.DMA,
        ],
    )


NR = NP * H // 128
BMP = 3200
NBLK_TC = NR // BMP
VALID_ROWS = N_NODES * H // 128


def _tc_first_body(x_r, d_r, m1_r, u_r, dv_r):
    dvp = lax.rsqrt(d_r[0] + d_r[1] + 1.0)
    xb = x_r[...]
    hparts = [jnp.dot(xb, m1_r[q], preferred_element_type=jnp.float32, precision=lax.Precision.HIGHEST)
              for q in range(4)]
    h = jnp.stack(hparts, axis=1).reshape(BMP, 128)
    u_r[...] = h * dvp
    dv_r[...] = dvp


def _tc_first(xp4, degp, m1):
    return pl.pallas_call(
        _tc_first_body,
        grid=(NBLK_TC,),
        in_specs=[
            pl.BlockSpec((BMP // 4, 128), lambda i: (i, 0)),
            pl.BlockSpec((NC, BMP, 128), lambda i: (0, i, 0)),
            pl.BlockSpec((4, 128, 128), lambda i: (0, 0, 0)),
        ],
        out_specs=[
            pl.BlockSpec((BMP, 128), lambda i: (i, 0)),
            pl.BlockSpec((BMP, 128), lambda i: (i, 0)),
        ],
        out_shape=[
            jax.ShapeDtypeStruct((NR, 128), jnp.float32),
            jax.ShapeDtypeStruct((NR, 128), jnp.float32),
        ],
    )(xp4, degp, m1)


def _tc_mid_body(s_r, u_r, dv_r, bt_r, wbd_r, o_r):
    i = pl.program_id(0)
    dvp = dv_r[...]
    a = jnp.maximum(dvp * (s_r[0] + s_r[1] + u_r[...]) + bt_r[...], 0.0)
    nxt = jnp.dot(a, wbd_r[...], preferred_element_type=jnp.float32, precision=lax.Precision.HIGHEST) * dvp
    row = lax.broadcasted_iota(jnp.int32, (BMP, 1), 0) + i * BMP
    o_r[...] = jnp.where(row < VALID_ROWS, nxt, 0.0)


def _tc_mid(sv, u, dvp, bt, wbd):
    return pl.pallas_call(
        _tc_mid_body,
        grid=(NBLK_TC,),
        in_specs=[
            pl.BlockSpec((NC, BMP, 128), lambda i: (0, i, 0)),
            pl.BlockSpec((BMP, 128), lambda i: (i, 0)),
            pl.BlockSpec((BMP, 128), lambda i: (i, 0)),
            pl.BlockSpec((1, 128), lambda i: (0, 0)),
            pl.BlockSpec((128, 128), lambda i: (0, 0)),
        ],
        out_specs=pl.BlockSpec((BMP, 128), lambda i: (i, 0)),
        out_shape=jax.ShapeDtypeStruct((NR, 128), jnp.float32),
    )(sv, u, dvp, bt, wbd)


def _tc_tail_body(s_r, u_r, dv_r, bt_r, b0_r, b1_r, b2_r, b3_r,
                  b4_r, b5_r, b6_r, b7_r, y_r, w1a_r, w1b_r, fb1_r, w2_r,
                  fb2_r, w3_r, fb3_r, o_r, pacc):
    i = pl.program_id(0)
    dvp = dv_r[...]
    a = jnp.maximum(dvp * (s_r[0] + s_r[1] + u_r[...]) + bt_r[...], 0.0)
    iog = lax.broadcasted_iota(jnp.int32, (BMP, G), 1)
    bq = (b0_r, b1_r, b2_r, b3_r, b4_r, b5_r, b6_r, b7_r)
    part = jnp.zeros((G, H), jnp.float32)
    for q in range(8):
        oh = (bq[q][...] == iog).astype(jnp.float32)
        part = part + lax.dot_general(oh, a[:, q * H:(q + 1) * H],
                                      (((0,), (0,)), ((), ())),
                                      preferred_element_type=jnp.float32, precision=lax.Precision.HIGHEST)

    @pl.when(i == 0)
    def _init():
        pacc[...] = part

    @pl.when(i > 0)
    def _accum():
        pacc[...] = pacc[...] + part

    @pl.when(i == NBLK_TC - 1)
    def _head():
        p = pacc[...]
        z = jnp.dot(p, w1a_r[...], preferred_element_type=jnp.float32, precision=lax.Precision.HIGHEST)
        z = z + jnp.dot(y_r[...], w1b_r[...], preferred_element_type=jnp.float32, precision=lax.Precision.HIGHEST)
        z = jnp.maximum(z + fb1_r[...], 0.0)
        z = jnp.maximum(jnp.dot(z, w2_r[...], preferred_element_type=jnp.float32, precision=lax.Precision.HIGHEST) + fb2_r[...], 0.0)
        o_r[...] = jnp.dot(z, w3_r[...], preferred_element_type=jnp.float32, precision=lax.Precision.HIGHEST) + fb3_r[...]


def _tc_tail(sv, u, dvp, bt, bqs, y, fc1_W, fc1_b, fc2_W, fc2_b, fc3_W, fc3_b):
    return pl.pallas_call(
        _tc_tail_body,
        grid=(NBLK_TC,),
        in_specs=[
            pl.BlockSpec((NC, BMP, 128), lambda i: (0, i, 0)),
            pl.BlockSpec((BMP, 128), lambda i: (i, 0)),
            pl.BlockSpec((BMP, 128), lambda i: (i, 0)),
            pl.BlockSpec((1, 128), lambda i: (0, 0)),
        ] + [pl.BlockSpec((BMP, 1), lambda i: (i, 0))] * 8 + [
            pl.BlockSpec((G, F), lambda i: (0, 0)),
            pl.BlockSpec((H, H), lambda i: (0, 0)),
            pl.BlockSpec((F, H), lambda i: (0, 0)),
            pl.BlockSpec((1, H), lambda i: (0, 0)),
            pl.BlockSpec((H, H), lambda i: (0, 0)),
            pl.BlockSpec((1, H), lambda i: (0, 0)),
            pl.BlockSpec((H, 1), lambda i: (0, 0)),
            pl.BlockSpec((1, 1), lambda i: (0, 0)),
        ],
        out_specs=pl.BlockSpec((G, 1), lambda i: (0, 0)),
        out_shape=jax.ShapeDtypeStruct((G, 1), jnp.float32),
        scratch_shapes=[pltpu.VMEM((G, H), jnp.float32)],
    )(sv, u, dvp, bt, *bqs, y, fc1_W[:H], fc1_W[H:],
      fc1_b[None], fc2_W, fc2_b[None], fc3_W, fc3_b[None])


def kernel(x, edge_index, y, batch, W1, b1, W2, b2, W3, b3,
           fc1_W, fc1_b, fc2_W, fc2_b, fc3_W, fc3_b):
    ei3 = edge_index.reshape(2, N_EDGES // CH, CH)
    ei3p = jnp.pad(ei3, ((0, 0), (0, (EPAD - N_EDGES) // CH), (0, 0)),
                   constant_values=N_NODES)
    dst2 = ei3p[1]
    edges2 = jnp.swapaxes(ei3p, 0, 1)
    xp4 = jnp.pad(x, ((0, NP - N_NODES), (0, 0))).reshape(NP * F // 128, 128)
    batch_p = jnp.pad(batch, (0, NP - N_NODES), constant_values=G)
    bqs = [batch_p[q::8][:, None] for q in range(8)]

    kk = jnp.arange(128)
    cc = jnp.arange(128)
    qq = jnp.arange(4)
    w1big = W1[(kk % F)[:, None], (cc % H)[None, :]]
    m1 = ((kk // F)[None, :, None] == 8 * qq[:, None, None]
          + cc[None, None, :] // H).astype(jnp.float32) * w1big[None]
    eye8 = jnp.eye(8, dtype=jnp.float32)
    w2bd = jnp.kron(eye8, W2)
    w3bd = jnp.kron(eye8, W3)
    b1t = jnp.tile(b1, 8)[None]
    b2t = jnp.tile(b2, 8)[None]
    b3t = jnp.tile(b3, 8)[None]

    deg = _sc_deg()(dst2)
    degp = deg.reshape(NC, NR, 128)
    u1, dvp = _tc_first(xp4, degp, m1)
    s1 = _sc_prop()(u1.reshape(NP, H), edges2)
    u2 = _tc_mid(s1.reshape(NC, NR, 128), u1, dvp, b1t, w2bd)
    s2 = _sc_prop()(u2.reshape(NP, H), edges2)
    u3 = _tc_mid(s2.reshape(NC, NR, 128), u2, dvp, b2t, w3bd)
    s3 = _sc_prop()(u3.reshape(NP, H), edges2)
    return _tc_tail(s3.reshape(NC, NR, 128), u3, dvp, b3t, bqs, y,
                    fc1_W, fc1_b, fc2_W, fc2_b, fc3_W, fc3_b)
